# baseline (speedup 1.0000x reference)
import functools

import jax
import jax.numpy as jnp
from jax import lax
from jax.experimental import pallas as pl
from jax.experimental.pallas import tpu as pltpu
from jax.experimental.pallas import tpu_sc as plsc

NC = 2
NS = 16
NW = NC * NS
LANES = 16


def _deg_kernel_fn(N, E, EPW):
    nvec = EPW // LANES
    SLAB = 624
    SLABW = 640

    mesh = plsc.VectorSubcoreMesh(core_axis_name="c", subcore_axis_name="s")

    @functools.partial(
        pl.kernel,
        mesh=mesh,
        out_type=jax.ShapeDtypeStruct((NC * N,), jnp.float32),
        scratch_types=[
            pltpu.VMEM((EPW,), jnp.int32),
            pltpu.VMEM((N,), jnp.float32),
            pltpu.VMEM((NS * SLABW,), jnp.float32),
            pltpu.VMEM((SLABW,), jnp.float32),
            pltpu.VMEM_SHARED((NS * N,), jnp.float32),
        ],
        compiler_params=pltpu.CompilerParams(needs_layout_passes=False),
    )
    def deg_kernel(edges_hbm, out_hbm, dst_all, deg_v, red_v, sum_v, degs):
        c = lax.axis_index("c")
        s = lax.axis_index("s")
        wid = s * NC + c

        zeros16 = jnp.zeros((LANES,), jnp.float32)

        def zero_body(i, _):
            deg_v[pl.ds(i * LANES, LANES)] = zeros16
            return 0

        lax.fori_loop(0, N // LANES, zero_body, 0)

        pltpu.sync_copy(edges_hbm.at[pl.ds(E + wid * EPW, EPW)], dst_all)

        ones16 = jnp.ones((LANES,), jnp.float32)

        def body(j, _):
            idx = dst_all[pl.ds(j * LANES, LANES)]
            plsc.addupdate_scatter(deg_v, [idx], ones16)
            return 0

        lax.fori_loop(0, nvec, body, 0)

        pltpu.sync_copy(deg_v, degs.at[pl.ds(s * N, N)])
        plsc.subcore_barrier()

        rb = s * SLAB
        for t in range(NS):
            pltpu.sync_copy(
                degs.at[pl.ds(t * N + rb, SLABW)],
                red_v.at[pl.ds(t * SLABW, SLABW)],
            )

        def red_body(g, _):
            v = zeros16
            for t in range(NS):
                v = v + red_v[pl.ds(t * SLABW + g * LANES, LANES)]
            sum_v[pl.ds(g * LANES, LANES)] = v
            return 0

        lax.fori_loop(0, SLABW // LANES, red_body, 0)

        pltpu.sync_copy(sum_v, out_hbm.at[pl.ds(c * N + rb, SLABW)])

    return deg_kernel


def _mlp_body(x_ref, p0_ref, p1_ref, w1_ref, b1_ref, w2_ref, b2_ref,
              h2_ref, dinv_ref):
    deg = p0_ref[...] + p1_ref[...]
    dinv = lax.rsqrt(jnp.maximum(deg, 1.0))
    h = jnp.maximum(
        jnp.dot(x_ref[...], w1_ref[...], preferred_element_type=jnp.float32)
        + b1_ref[...],
        0.0,
    )
    h2 = (
        jnp.dot(h, w2_ref[...], preferred_element_type=jnp.float32) + b2_ref[...]
    ) * dinv
    h2_ref[...] = h2
    dinv_ref[...] = dinv


def _spmm_kernel_fn(N, D, E, EPW, K):
    NCH = EPW // K
    TAIL = EPW - NCH * K
    SLAB = 624
    SLABW = 640

    mesh = plsc.VectorSubcoreMesh(core_axis_name="c", subcore_axis_name="s")

    @functools.partial(
        pl.kernel,
        mesh=mesh,
        out_type=jax.ShapeDtypeStruct((NC, N, D), jnp.float32),
        scratch_types=[
            pltpu.VMEM((EPW,), jnp.int32),
            pltpu.VMEM((EPW,), jnp.int32),
            pltpu.VMEM((K, D), jnp.float32),
            pltpu.VMEM((K, D), jnp.float32),
            pltpu.VMEM_SHARED((N, D), jnp.float32),
            pltpu.SemaphoreType.DMA,
            pltpu.SemaphoreType.DMA,
        ],
        compiler_params=pltpu.CompilerParams(needs_layout_passes=False),
    )
    def spmm_kernel(edges_hbm, h2_hbm, out_hbm, src_all, dst_all,
                    rows0, rows1, acc, sem0, sem1):
        c = lax.axis_index("c")
        s = lax.axis_index("s")
        wid = s * NC + c
        ebase = wid * EPW
        rbase = s * SLAB

        zeros16 = jnp.zeros((LANES,), jnp.float32)

        def zrow(i, _):
            for k in range(D // LANES):
                rows0[i, pl.ds(k * LANES, LANES)] = zeros16
            return 0

        lax.fori_loop(0, K, zrow, 0)

        def zcopy(j, _):
            pltpu.sync_copy(rows0, acc.at[pl.ds(rbase + j * K, K)])
            return 0

        lax.fori_loop(0, SLABW // K, zcopy, 0)
        ZREM = SLABW - (SLABW // K) * K
        if ZREM:
            pltpu.sync_copy(
                rows0.at[pl.ds(0, ZREM)],
                acc.at[pl.ds(rbase + SLABW - ZREM, ZREM)],
            )

        pltpu.sync_copy(edges_hbm.at[pl.ds(ebase, EPW)], src_all)
        pltpu.sync_copy(edges_hbm.at[pl.ds(E + ebase, EPW)], dst_all)

        plsc.subcore_barrier()

        def gather(j, buf, sem):
            pltpu.async_copy(h2_hbm.at[src_all.at[pl.ds(j * K, K)]], buf, sem)

        def scatter(j, buf):
            pltpu.sync_copy(buf, acc.at[dst_all.at[pl.ds(j * K, K)]], add=True)

        gather(0, rows0, sem0)

        def chunk(j, _):
            def step(buf, sem, obuf, osem):
                pltpu.make_async_copy(h2_hbm.at[pl.ds(0, K)], buf, sem).wait()

                @pl.when(j + 1 < NCH)
                def _():
                    gather(j + 1, obuf, osem)

                scatter(j, buf)

            @pl.when(j % 2 == 0)
            def _():
                step(rows0, sem0, rows1, sem1)

            @pl.when(j % 2 == 1)
            def _():
                step(rows1, sem1, rows0, sem0)

            return 0

        lax.fori_loop(0, NCH, chunk, 0)

        if TAIL:
            tbase = NCH * K
            pltpu.async_copy(
                h2_hbm.at[src_all.at[pl.ds(tbase, TAIL)]],
                rows0.at[pl.ds(0, TAIL)],
                sem0,
            ).wait()
            pltpu.sync_copy(
                rows0.at[pl.ds(0, TAIL)],
                acc.at[dst_all.at[pl.ds(tbase, TAIL)]],
                add=True,
            )

        plsc.subcore_barrier()

        pltpu.sync_copy(
            acc.at[pl.ds(rbase, SLABW)],
            out_hbm.at[c, pl.ds(rbase, SLABW)],
        )

    return spmm_kernel


def _final_body(p_ref, dinv_ref, out_ref):
    v = (p_ref[0] + p_ref[1]) * dinv_ref[...]
    m = jnp.max(v, axis=1, keepdims=True)
    e = jnp.exp(v - m)
    lse = jnp.log(jnp.sum(e, axis=1, keepdims=True)) + m
    out_ref[...] = v - lse


def kernel(x, edge_index, W1, b1, W2, b2):
    N, D_IN = x.shape
    D_H = W1.shape[1]
    D = W2.shape[1]
    E = edge_index.shape[1]

    EPW = E // NW
    K = 112

    edges = edge_index.reshape(2 * E)

    degp = _deg_kernel_fn(N, E, EPW)(edges)
    p0 = degp[:N].reshape(N, 1)
    p1 = degp[N:].reshape(N, 1)

    BN = 400
    grid = N // BN
    h2, dinv = pl.pallas_call(
        _mlp_body,
        grid=(grid,),
        in_specs=[
            pl.BlockSpec((BN, D_IN), lambda i: (i, 0)),
            pl.BlockSpec((BN, 1), lambda i: (i, 0)),
            pl.BlockSpec((BN, 1), lambda i: (i, 0)),
            pl.BlockSpec((D_IN, D_H), lambda i: (0, 0)),
            pl.BlockSpec((1, D_H), lambda i: (0, 0)),
            pl.BlockSpec((D_H, D), lambda i: (0, 0)),
            pl.BlockSpec((1, D), lambda i: (0, 0)),
        ],
        out_specs=[
            pl.BlockSpec((BN, D), lambda i: (i, 0)),
            pl.BlockSpec((BN, 1), lambda i: (i, 0)),
        ],
        out_shape=[
            jax.ShapeDtypeStruct((N, D), jnp.float32),
            jax.ShapeDtypeStruct((N, 1), jnp.float32),
        ],
    )(x, p0, p1, W1, b1.reshape(1, D_H), W2, b2.reshape(1, D))

    partials = _spmm_kernel_fn(N, D, E, EPW, K)(edges, h2)

    out = pl.pallas_call(
        _final_body,
        grid=(grid,),
        in_specs=[
            pl.BlockSpec((NC, BN, D), lambda i: (0, i, 0)),
            pl.BlockSpec((BN, 1), lambda i: (i, 0)),
        ],
        out_specs=pl.BlockSpec((BN, D), lambda i: (i, 0)),
        out_shape=jax.ShapeDtypeStruct((N, D), jnp.float32),
    )(partials, dinv)

    return out


# speedup vs baseline: 1.1137x; 1.1137x over previous
import functools

import jax
import jax.numpy as jnp
from jax import lax
from jax.experimental import pallas as pl
from jax.experimental.pallas import tpu as pltpu
from jax.experimental.pallas import tpu_sc as plsc

NC = 2
NS = 16
NW = NC * NS
LANES = 16


def _deg_kernel_fn(N, E, EPW):
    nvec = EPW // LANES
    SLAB = 624
    SLABW = 640

    mesh = plsc.VectorSubcoreMesh(core_axis_name="c", subcore_axis_name="s")

    @functools.partial(
        pl.kernel,
        mesh=mesh,
        out_type=jax.ShapeDtypeStruct((NC * N,), jnp.float32),
        scratch_types=[
            pltpu.VMEM((EPW,), jnp.int32),
            pltpu.VMEM((N,), jnp.float32),
            pltpu.VMEM((NS * SLABW,), jnp.float32),
            pltpu.VMEM((SLABW,), jnp.float32),
            pltpu.VMEM_SHARED((NS * N,), jnp.float32),
        ],
        compiler_params=pltpu.CompilerParams(needs_layout_passes=False),
    )
    def deg_kernel(edges_hbm, out_hbm, dst_all, deg_v, red_v, sum_v, degs):
        c = lax.axis_index("c")
        s = lax.axis_index("s")
        wid = s * NC + c

        zeros16 = jnp.zeros((LANES,), jnp.float32)

        def zero_body(i, _):
            deg_v[pl.ds(i * LANES, LANES)] = zeros16
            return 0

        lax.fori_loop(0, N // LANES, zero_body, 0)

        pltpu.sync_copy(edges_hbm.at[pl.ds(E + wid * EPW, EPW)], dst_all)

        ones16 = jnp.ones((LANES,), jnp.float32)

        def body(j, _):
            idx = dst_all[pl.ds(j * LANES, LANES)]
            plsc.addupdate_scatter(deg_v, [idx], ones16)
            return 0

        lax.fori_loop(0, nvec, body, 0)

        pltpu.sync_copy(deg_v, degs.at[pl.ds(s * N, N)])
        plsc.subcore_barrier()

        rb = s * SLAB
        for t in range(NS):
            pltpu.sync_copy(
                degs.at[pl.ds(t * N + rb, SLABW)],
                red_v.at[pl.ds(t * SLABW, SLABW)],
            )

        def red_body(g, _):
            v = zeros16
            for t in range(NS):
                v = v + red_v[pl.ds(t * SLABW + g * LANES, LANES)]
            sum_v[pl.ds(g * LANES, LANES)] = v
            return 0

        lax.fori_loop(0, SLABW // LANES, red_body, 0)

        pltpu.sync_copy(sum_v, out_hbm.at[pl.ds(c * N + rb, SLABW)])

    return deg_kernel


def _mlp_body(x_ref, p0_ref, p1_ref, w1_ref, b1_ref, w2_ref, b2_ref,
              h2_ref, dinv_ref):
    deg = p0_ref[...] + p1_ref[...]
    dinv = lax.rsqrt(jnp.maximum(deg, 1.0))
    h = jnp.maximum(
        jnp.dot(x_ref[...], w1_ref[...], preferred_element_type=jnp.float32)
        + b1_ref[...],
        0.0,
    )
    h2 = (
        jnp.dot(h, w2_ref[...], preferred_element_type=jnp.float32) + b2_ref[...]
    ) * dinv
    h2_ref[...] = h2
    dinv_ref[...] = dinv


def _spmm_kernel_fn(N, D, E, EPW, K):
    NCH = EPW // K
    TAIL = EPW - NCH * K
    SLAB = 624
    SLABW = 640

    mesh = plsc.VectorSubcoreMesh(core_axis_name="c", subcore_axis_name="s")

    @functools.partial(
        pl.kernel,
        mesh=mesh,
        out_type=jax.ShapeDtypeStruct((NC, N, D), jnp.float32),
        scratch_types=[
            pltpu.VMEM((EPW,), jnp.int32),
            pltpu.VMEM((EPW,), jnp.int32),
            pltpu.VMEM((K, D), jnp.float32),
            pltpu.VMEM((K, D), jnp.float32),
            pltpu.VMEM_SHARED((N, D), jnp.float32),
            pltpu.SemaphoreType.DMA,
            pltpu.SemaphoreType.DMA,
        ],
        compiler_params=pltpu.CompilerParams(needs_layout_passes=False),
    )
    def spmm_kernel(edges_hbm, h2_hbm, out_hbm, src_all, dst_all,
                    rows0, rows1, acc, sem0, sem1):
        c = lax.axis_index("c")
        s = lax.axis_index("s")
        wid = s * NC + c
        ebase = wid * EPW
        rbase = s * SLAB

        zeros16 = jnp.zeros((LANES,), jnp.float32)

        def zrow(i, _):
            for k in range(D // LANES):
                rows0[i, pl.ds(k * LANES, LANES)] = zeros16
            return 0

        lax.fori_loop(0, K, zrow, 0)

        def zcopy(j, _):
            pltpu.sync_copy(rows0, acc.at[pl.ds(rbase + j * K, K)])
            return 0

        lax.fori_loop(0, SLABW // K, zcopy, 0)
        ZREM = SLABW - (SLABW // K) * K
        if ZREM:
            pltpu.sync_copy(
                rows0.at[pl.ds(0, ZREM)],
                acc.at[pl.ds(rbase + SLABW - ZREM, ZREM)],
            )

        pltpu.sync_copy(edges_hbm.at[pl.ds(ebase, EPW)], src_all)
        pltpu.sync_copy(edges_hbm.at[pl.ds(E + ebase, EPW)], dst_all)

        plsc.subcore_barrier()

        def gather(j, buf, sem):
            pltpu.async_copy(h2_hbm.at[src_all.at[pl.ds(j * K, K)]], buf, sem)

        def scatter(j, buf):
            pltpu.sync_copy(buf, acc.at[dst_all.at[pl.ds(j * K, K)]], add=True)

        gather(0, rows0, sem0)

        def chunk(j, _):
            def step(buf, sem, obuf, osem):
                pltpu.make_async_copy(h2_hbm.at[pl.ds(0, K)], buf, sem).wait()

                @pl.when(j + 1 < NCH)
                def _():
                    gather(j + 1, obuf, osem)

                scatter(j, buf)

            @pl.when(j % 2 == 0)
            def _():
                step(rows0, sem0, rows1, sem1)

            @pl.when(j % 2 == 1)
            def _():
                step(rows1, sem1, rows0, sem0)

            return 0

        lax.fori_loop(0, NCH, chunk, 0)

        if TAIL:
            tbase = NCH * K
            pltpu.async_copy(
                h2_hbm.at[src_all.at[pl.ds(tbase, TAIL)]],
                rows0.at[pl.ds(0, TAIL)],
                sem0,
            ).wait()
            pltpu.sync_copy(
                rows0.at[pl.ds(0, TAIL)],
                acc.at[dst_all.at[pl.ds(tbase, TAIL)]],
                add=True,
            )

        plsc.subcore_barrier()

        pltpu.sync_copy(
            acc.at[pl.ds(rbase, SLABW)],
            out_hbm.at[c, pl.ds(rbase, SLABW)],
        )

    return spmm_kernel


def _final_body(p_ref, dinv_ref, out_ref):
    v = (p_ref[0] + p_ref[1]) * dinv_ref[...]
    m = jnp.max(v, axis=1, keepdims=True)
    e = jnp.exp(v - m)
    lse = jnp.log(jnp.sum(e, axis=1, keepdims=True)) + m
    out_ref[...] = v - lse


def kernel(x, edge_index, W1, b1, W2, b2):
    N, D_IN = x.shape
    D_H = W1.shape[1]
    D = W2.shape[1]
    E = edge_index.shape[1]

    EPW = E // NW
    K = 112

    edges = edge_index.reshape(2 * E)

    degp = _deg_kernel_fn(N, E, EPW)(edges)
    p0 = degp[:N].reshape(N, 1)
    p1 = degp[N:].reshape(N, 1)

    BN = 2000
    grid = N // BN
    h2, dinv = pl.pallas_call(
        _mlp_body,
        grid=(grid,),
        in_specs=[
            pl.BlockSpec((BN, D_IN), lambda i: (i, 0)),
            pl.BlockSpec((BN, 1), lambda i: (i, 0)),
            pl.BlockSpec((BN, 1), lambda i: (i, 0)),
            pl.BlockSpec((D_IN, D_H), lambda i: (0, 0)),
            pl.BlockSpec((1, D_H), lambda i: (0, 0)),
            pl.BlockSpec((D_H, D), lambda i: (0, 0)),
            pl.BlockSpec((1, D), lambda i: (0, 0)),
        ],
        out_specs=[
            pl.BlockSpec((BN, D), lambda i: (i, 0)),
            pl.BlockSpec((BN, 1), lambda i: (i, 0)),
        ],
        out_shape=[
            jax.ShapeDtypeStruct((N, D), jnp.float32),
            jax.ShapeDtypeStruct((N, 1), jnp.float32),
        ],
    )(x, p0, p1, W1, b1.reshape(1, D_H), W2, b2.reshape(1, D))

    partials = _spmm_kernel_fn(N, D, E, EPW, K)(edges, h2)

    out = pl.pallas_call(
        _final_body,
        grid=(grid,),
        in_specs=[
            pl.BlockSpec((NC, BN, D), lambda i: (0, i, 0)),
            pl.BlockSpec((BN, 1), lambda i: (i, 0)),
        ],
        out_specs=pl.BlockSpec((BN, D), lambda i: (i, 0)),
        out_shape=jax.ShapeDtypeStruct((N, D), jnp.float32),
    )(partials, dinv)

    return out
